# baseline (speedup 1.0000x reference)
import functools

import jax
import jax.numpy as jnp
from jax import lax
from jax.experimental import pallas as pl
from jax.experimental.pallas import tpu as pltpu
from jax.experimental.pallas import tpu_sc as plsc

L = 16
GN = 8
NW = 32
S = 49


@functools.lru_cache(maxsize=None)
def _build(B, NSAMP, N, D):
    R = GN * NSAMP
    RC = R + GN
    W = 2 * R + GN
    assert B % GN == 0 and D % L == 0
    assert R % 8 == 0 and RC % 8 == 0 and W % 8 == 0
    NG = B // GN
    GPW = -(-NG // NW)
    assert NG % 2 == 0 and GPW % 2 == 0 and NG > (NW - 1) * GPW
    assert NG >= S
    KV = D // L
    inv = jnp.float32(1.0 / NSAMP)

    mesh = plsc.VectorSubcoreMesh(core_axis_name="c", subcore_axis_name="s")

    @functools.partial(
        pl.kernel,
        mesh=mesh,
        out_type=[
            jax.ShapeDtypeStruct((B, 3 * D), jnp.float32),
            jax.ShapeDtypeStruct((B, 3 * D), jnp.float32),
        ],
        scratch_types=[
            pltpu.VMEM((2 * S * W,), jnp.int32),
            pltpu.VMEM((R, D), jnp.float32),
            pltpu.VMEM((R, D), jnp.float32),
            pltpu.VMEM((RC, D), jnp.float32),
            pltpu.VMEM((RC, D), jnp.float32),
            pltpu.VMEM((R, D), jnp.float32),
            pltpu.VMEM((R, D), jnp.float32),
            pltpu.VMEM((RC, D), jnp.float32),
            pltpu.VMEM((RC, D), jnp.float32),
            pltpu.VMEM((GN, 3 * D), jnp.float32),
            pltpu.VMEM((GN, 3 * D), jnp.float32),
            pltpu.VMEM((GN, 3 * D), jnp.float32),
            pltpu.VMEM((GN, 3 * D), jnp.float32),
            pltpu.SemaphoreType.DMA,
            pltpu.SemaphoreType.DMA,
            pltpu.SemaphoreType.DMA,
            pltpu.SemaphoreType.DMA,
        ],
    )
    def agg(nidx, fb, fu, outb, outu,
            idx,
            rpb0, rpu0, rnb0, rnu0,
            rpb1, rpu1, rnb1, rnu1,
            ob0, ou0, ob1, ou1,
            gsem0, gsem1, osem0, osem1):
        c = lax.axis_index("c")
        s = lax.axis_index("s")
        wid = s * 2 + c
        lo = wid * GPW
        hi = jnp.minimum(lo + GPW, NG)
        T = (hi - lo) // 2

        RPB = [rpb0, rpb1]
        RPU = [rpu0, rpu1]
        RNB = [rnb0, rnb1]
        RNU = [rnu0, rnu1]
        OB = [ob0, ob1]
        OU = [ou0, ou1]
        GSEM = [gsem0, gsem1]
        OSEM = [osem0, osem1]

        def chunk_of(g):
            grel = g - lo
            cidx = grel // S
            cs = jnp.minimum(lo + cidx * S, NG - S)
            return cidx, cs

        def stage(cp, cs):
            pltpu.sync_copy(nidx.at[pl.ds(cs * W, S * W)],
                            idx.at[pl.ds(cp * S * W, S * W)])

        def maybe_stage(g):
            grel = g - lo
            cidx, cs = chunk_of(g)

            @pl.when(grel % S == 0)
            def _():
                stage(cidx % 2, cs)

        def idx_slices(g):
            cidx, cs = chunk_of(g)
            j = (cidx % 2) * S + (g - cs)
            return (idx.at[pl.ds(j * W, R)],
                    idx.at[pl.ds(j * W + R, RC)])

        def gather_copies(p, g):
            ip, inc = idx_slices(g)
            sem = GSEM[p]
            return [
                (fb.at[ip], RPB[p], sem),
                (fu.at[ip], RPU[p], sem),
                (fb.at[inc], RNB[p], sem),
                (fu.at[inc], RNU[p], sem),
            ]

        def issue(p, g):
            for src, dst, sem in gather_copies(p, g):
                pltpu.async_copy(src, dst, sem)

        def wait_gathers(p, g):
            for src, dst, sem in gather_copies(p, g):
                pltpu.make_async_copy(src, dst, sem).wait()

        def out_copies(p, g):
            rows = pl.ds(g * GN, GN)
            return [
                (OB[p], outb.at[rows], OSEM[p]),
                (OU[p], outu.at[rows], OSEM[p]),
            ]

        def issue_out(p, g):
            for src, dst, sem in out_copies(p, g):
                pltpu.async_copy(src, dst, sem)

        def wait_out(p, g):
            for src, dst, sem in out_copies(p, g):
                pltpu.make_async_copy(src, dst, sem).wait()

        def reduce_group(p):
            rpb, rpu = RPB[p], RPU[p]
            rnb, rnu = RNB[p], RNU[p]
            ob, ou = OB[p], OU[p]

            def node(r, _):
                base = r * NSAMP
                for rows, out, col0 in (
                        (rpb, ob, 0), (rnu, ob, D),
                        (rpu, ou, 0), (rnb, ou, D)):
                    acc = [rows[base, pl.ds(k * L, L)] for k in range(KV)]
                    for j in range(1, NSAMP):
                        acc = [acc[k] + rows[base + j, pl.ds(k * L, L)]
                               for k in range(KV)]
                    for k in range(KV):
                        out[r, pl.ds(col0 + k * L, L)] = acc[k] * inv
                for rows, out in ((rnb, ob), (rnu, ou)):
                    for k in range(KV):
                        out[r, pl.ds(2 * D + k * L, L)] = rows[R + r, pl.ds(k * L, L)]
                return 0

            lax.fori_loop(0, GN, node, 0)

        stage(0, lo)
        issue(0, lo)

        def body(t, _):
            g0 = lo + 2 * t
            g1 = g0 + 1
            g2 = g0 + 2

            maybe_stage(g1)
            issue(1, g1)

            wait_gathers(0, g0)

            @pl.when(g0 - lo >= 2)
            def _():
                wait_out(0, g0 - 2)

            reduce_group(0)
            issue_out(0, g0)

            @pl.when(g2 < hi)
            def _():
                maybe_stage(g2)
                issue(0, g2)

            wait_gathers(1, g1)

            @pl.when(g1 - lo >= 2)
            def _():
                wait_out(1, g1 - 2)

            reduce_group(1)
            issue_out(1, g1)
            return 0

        lax.fori_loop(0, T, body, 0)

        wait_out(0, hi - 2)
        wait_out(1, hi - 1)

    return agg


def kernel(nodes, neighs_pos, neighs_neg, feat_bal, feat_unbal):
    B, NSAMP = neighs_pos.shape
    N, D = feat_bal.shape
    agg = _build(B, NSAMP, N, D)
    NG = B // GN
    nidx = jnp.concatenate(
        [neighs_pos.astype(jnp.int32).reshape(NG, GN * NSAMP),
         neighs_neg.astype(jnp.int32).reshape(NG, GN * NSAMP),
         nodes.astype(jnp.int32).reshape(NG, GN)], axis=1)
    out_bal, out_unbal = agg(
        nidx.reshape(-1),
        feat_bal,
        feat_unbal,
    )
    return out_bal, out_unbal


# speedup vs baseline: 1.0190x; 1.0190x over previous
import functools

import jax
import jax.numpy as jnp
from jax import lax
from jax.experimental import pallas as pl
from jax.experimental.pallas import tpu as pltpu
from jax.experimental.pallas import tpu_sc as plsc

L = 16
GN = 8
NW = 32
S = 49


@functools.lru_cache(maxsize=None)
def _build(B, NSAMP, N, D):
    R = GN * NSAMP
    assert B % GN == 0 and D % L == 0
    NG = B // GN
    GPW = -(-NG // NW)
    assert NG % 2 == 0 and GPW % 2 == 0 and NG > (NW - 1) * GPW
    assert NG >= S
    KV = D // L
    inv = jnp.float32(1.0 / NSAMP)

    mesh = plsc.VectorSubcoreMesh(core_axis_name="c", subcore_axis_name="s")

    @functools.partial(
        pl.kernel,
        mesh=mesh,
        out_type=[
            jax.ShapeDtypeStruct((B, 3 * D), jnp.float32),
            jax.ShapeDtypeStruct((B, 3 * D), jnp.float32),
        ],
        scratch_types=[
            pltpu.VMEM((3 * S * R,), jnp.int32),
            pltpu.VMEM((3 * S * R,), jnp.int32),
            pltpu.VMEM((3 * S * GN,), jnp.int32),
            pltpu.VMEM((R, D), jnp.float32),
            pltpu.VMEM((R, D), jnp.float32),
            pltpu.VMEM((R, D), jnp.float32),
            pltpu.VMEM((R, D), jnp.float32),
            pltpu.VMEM((GN, D), jnp.float32),
            pltpu.VMEM((GN, D), jnp.float32),
            pltpu.VMEM((R, D), jnp.float32),
            pltpu.VMEM((R, D), jnp.float32),
            pltpu.VMEM((R, D), jnp.float32),
            pltpu.VMEM((R, D), jnp.float32),
            pltpu.VMEM((GN, D), jnp.float32),
            pltpu.VMEM((GN, D), jnp.float32),
            pltpu.VMEM((GN, 3 * D), jnp.float32),
            pltpu.VMEM((GN, 3 * D), jnp.float32),
            pltpu.VMEM((GN, 3 * D), jnp.float32),
            pltpu.VMEM((GN, 3 * D), jnp.float32),
            pltpu.SemaphoreType.DMA,
            pltpu.SemaphoreType.DMA,
            pltpu.SemaphoreType.DMA,
            pltpu.SemaphoreType.DMA,
            pltpu.SemaphoreType.DMA,
        ],
    )
    def agg(npos2d, nneg2d, nodes2d, fb, fu, outb, outu,
            idxp3, idxn3, idxs3,
            rpb0, rpu0, rnb0, rnu0, rsb0, rsu0,
            rpb1, rpu1, rnb1, rnu1, rsb1, rsu1,
            ob0, ou0, ob1, ou1,
            gsem0, gsem1, osem0, osem1, isem):
        c = lax.axis_index("c")
        s = lax.axis_index("s")
        wid = s * 2 + c
        lo = wid * GPW
        hi = jnp.minimum(lo + GPW, NG)
        T = (hi - lo) // 2

        RPB = [rpb0, rpb1]
        RPU = [rpu0, rpu1]
        RNB = [rnb0, rnb1]
        RNU = [rnu0, rnu1]
        RSB = [rsb0, rsb1]
        RSU = [rsu0, rsu1]
        OB = [ob0, ob1]
        OU = [ou0, ou1]
        GSEM = [gsem0, gsem1]
        OSEM = [osem0, osem1]

        def chunk_of(g):
            grel = g - lo
            cidx = grel // S
            cs = jnp.minimum(lo + cidx * S, NG - S)
            return cidx, cs

        def stage_copies(cidx):
            cs = jnp.minimum(lo + cidx * S, NG - S)
            cp = cidx % 3
            return [
                (npos2d.at[pl.ds(cs * R, S * R)],
                 idxp3.at[pl.ds(cp * S * R, S * R)], isem),
                (nneg2d.at[pl.ds(cs * R, S * R)],
                 idxn3.at[pl.ds(cp * S * R, S * R)], isem),
                (nodes2d.at[pl.ds(cs * GN, S * GN)],
                 idxs3.at[pl.ds(cp * S * GN, S * GN)], isem),
            ]

        def issue_stage(cidx):
            @pl.when(lo + cidx * S < hi)
            def _():
                for src, dst, sem in stage_copies(cidx):
                    pltpu.async_copy(src, dst, sem)

        def wait_stage(cidx):
            for src, dst, sem in stage_copies(cidx):
                pltpu.make_async_copy(src, dst, sem).wait()

        def maybe_stage(g):
            grel = g - lo

            @pl.when(grel % S == 0)
            def _():
                cidx = grel // S
                wait_stage(cidx)
                issue_stage(cidx + 1)

        def idx_slices(g):
            cidx, cs = chunk_of(g)
            cp = cidx % 3
            j = cp * S + (g - cs)
            return (idxp3.at[pl.ds(j * R, R)],
                    idxn3.at[pl.ds(j * R, R)],
                    idxs3.at[pl.ds(j * GN, GN)])

        def gather_copies(p, g):
            ip, inn, isf = idx_slices(g)
            sem = GSEM[p]
            return [
                (fb.at[ip], RPB[p], sem),
                (fu.at[ip], RPU[p], sem),
                (fb.at[inn], RNB[p], sem),
                (fu.at[inn], RNU[p], sem),
                (fb.at[isf], RSB[p], sem),
                (fu.at[isf], RSU[p], sem),
            ]

        def issue(p, g):
            for src, dst, sem in gather_copies(p, g):
                pltpu.async_copy(src, dst, sem)

        def wait_gathers(p, g):
            for src, dst, sem in gather_copies(p, g):
                pltpu.make_async_copy(src, dst, sem).wait()

        def out_copies(p, g):
            rows = pl.ds(g * GN, GN)
            return [
                (OB[p], outb.at[rows], OSEM[p]),
                (OU[p], outu.at[rows], OSEM[p]),
            ]

        def issue_out(p, g):
            for src, dst, sem in out_copies(p, g):
                pltpu.async_copy(src, dst, sem)

        def wait_out(p, g):
            for src, dst, sem in out_copies(p, g):
                pltpu.make_async_copy(src, dst, sem).wait()

        def reduce_group(p):
            rpb, rpu = RPB[p], RPU[p]
            rnb, rnu = RNB[p], RNU[p]
            rsb, rsu = RSB[p], RSU[p]
            ob, ou = OB[p], OU[p]

            def node(r, _):
                base = r * NSAMP
                for rows, out, col0 in (
                        (rpb, ob, 0), (rnu, ob, D),
                        (rpu, ou, 0), (rnb, ou, D)):
                    acc = [rows[base, pl.ds(k * L, L)] for k in range(KV)]
                    for j in range(1, NSAMP):
                        acc = [acc[k] + rows[base + j, pl.ds(k * L, L)]
                               for k in range(KV)]
                    for k in range(KV):
                        out[r, pl.ds(col0 + k * L, L)] = acc[k] * inv
                for rows, out in ((rsb, ob), (rsu, ou)):
                    for k in range(KV):
                        out[r, pl.ds(2 * D + k * L, L)] = rows[r, pl.ds(k * L, L)]
                return 0

            lax.fori_loop(0, GN, node, 0)

        issue_stage(0)
        wait_stage(0)
        issue_stage(1)
        issue(0, lo)

        def body(t, _):
            g0 = lo + 2 * t
            g1 = g0 + 1
            g2 = g0 + 2

            maybe_stage(g1)
            issue(1, g1)

            wait_gathers(0, g0)

            @pl.when(g0 - lo >= 2)
            def _():
                wait_out(0, g0 - 2)

            reduce_group(0)
            issue_out(0, g0)

            @pl.when(g2 < hi)
            def _():
                maybe_stage(g2)
                issue(0, g2)

            wait_gathers(1, g1)

            @pl.when(g1 - lo >= 2)
            def _():
                wait_out(1, g1 - 2)

            reduce_group(1)
            issue_out(1, g1)
            return 0

        lax.fori_loop(0, T, body, 0)

        wait_out(0, hi - 2)
        wait_out(1, hi - 1)

    return agg


def kernel(nodes, neighs_pos, neighs_neg, feat_bal, feat_unbal):
    B, NSAMP = neighs_pos.shape
    N, D = feat_bal.shape
    agg = _build(B, NSAMP, N, D)
    NG = B // GN
    out_bal, out_unbal = agg(
        neighs_pos.reshape(-1).astype(jnp.int32),
        neighs_neg.reshape(-1).astype(jnp.int32),
        nodes.astype(jnp.int32),
        feat_bal,
        feat_unbal,
    )
    return out_bal, out_unbal
